# baseline (speedup 1.0000x reference)
import functools

import jax
import jax.numpy as jnp
from jax import lax
from jax.experimental import pallas as pl
from jax.experimental.pallas import tpu as pltpu
from jax.experimental.pallas import tpu_sc as plsc

HIDDEN = 128
NC = 2
NS = 16
NW = NC * NS
G = 128


NBUF = 2
P = 2


@functools.lru_cache(maxsize=None)
def _make_lookup(n_rows: int, hidden: int):
    rows_per_w = n_rows // NW
    ng = rows_per_w // G
    nblk = ng // P
    n_outer = nblk // NBUF
    mesh = plsc.VectorSubcoreMesh(
        core_axis_name="c", subcore_axis_name="s", num_cores=NC, num_subcores=NS
    )

    @functools.partial(
        pl.kernel,
        mesh=mesh,
        out_type=jax.ShapeDtypeStruct((n_rows, hidden), jnp.float32),
        scratch_types=[
            pltpu.VMEM((ng, G), jnp.int32),
            pltpu.VMEM((NBUF, P * G, hidden), jnp.float32),
            [pltpu.SemaphoreType.DMA] * NBUF,
        ],
    )
    def lookup(ids_hbm, table_hbm, out_hbm, idx_v, rows_v, sems):
        wid = lax.axis_index("s") * NC + lax.axis_index("c")
        base = wid * rows_per_w
        pltpu.sync_copy(ids_hbm.at[wid], idx_v)

        def gather(s, b):
            for p in range(P):
                pltpu.async_copy(
                    table_hbm.at[idx_v.at[s * P + p]],
                    rows_v.at[b].at[pl.ds(p * G, G)],
                    sems[b],
                )

        def gather_wait(b):
            pltpu.make_async_copy(
                out_hbm.at[pl.ds(0, P * G)], rows_v.at[b], sems[b]
            ).wait()

        def scatter(s, b):
            pltpu.sync_copy(
                rows_v.at[b], out_hbm.at[pl.ds(base + s * P * G, P * G)]
            )

        for b in range(NBUF):
            gather(b, b)

        def outer(so, carry):
            s0 = so * NBUF
            for b in range(NBUF):
                gather_wait(b)
                scatter(s0 + b, b)
                gather(s0 + b + NBUF, b)
            return carry

        lax.fori_loop(0, n_outer - 1, outer, 0)

        s0 = (n_outer - 1) * NBUF
        for b in range(NBUF):
            gather_wait(b)
            scatter(s0 + b, b)

    return lookup


def kernel(input_ids, embed_table):
    b, l = input_ids.shape
    n = b * l
    ids = input_ids.astype(jnp.int32).reshape(NW, n // (NW * G), G)
    out = _make_lookup(n, embed_table.shape[1])(ids, embed_table)
    return out.reshape(b, l, HIDDEN)


# speedup vs baseline: 1.0078x; 1.0078x over previous
import functools

import jax
import jax.numpy as jnp
from jax import lax
from jax.experimental import pallas as pl
from jax.experimental.pallas import tpu as pltpu
from jax.experimental.pallas import tpu_sc as plsc

HIDDEN = 128
NC = 2
NS = 16
NW = NC * NS
G = 128
NBUF = 5


@functools.lru_cache(maxsize=None)
def _make_lookup(n_rows: int, hidden: int):
    rows_per_w = n_rows // NW
    ng = rows_per_w // G
    n_outer = ng // NBUF
    mesh = plsc.VectorSubcoreMesh(
        core_axis_name="c", subcore_axis_name="s", num_cores=NC, num_subcores=NS
    )

    @functools.partial(
        pl.kernel,
        mesh=mesh,
        out_type=jax.ShapeDtypeStruct((n_rows, hidden), jnp.float32),
        scratch_types=[
            pltpu.VMEM((ng, G), jnp.int32),
            pltpu.VMEM((NBUF, G, hidden), jnp.float32),
            [pltpu.SemaphoreType.DMA] * NBUF,
        ],
    )
    def lookup(ids_hbm, table_hbm, out_hbm, idx_v, rows_v, sems):
        wid = lax.axis_index("s") * NC + lax.axis_index("c")
        base = wid * rows_per_w
        pltpu.sync_copy(ids_hbm.at[wid], idx_v)

        def gather(g, b):
            pltpu.async_copy(table_hbm.at[idx_v.at[g]], rows_v.at[b], sems[b])

        def gather_wait(b):
            pltpu.make_async_copy(
                table_hbm.at[idx_v.at[0]], rows_v.at[b], sems[b]
            ).wait()

        def scatter(g, b):
            pltpu.sync_copy(rows_v.at[b], out_hbm.at[pl.ds(base + g * G, G)])

        for b in range(NBUF):
            gather(b, b)

        def outer(go, carry):
            g0 = go * NBUF
            for b in range(NBUF):
                gather_wait(b)
                scatter(g0 + b, b)
                gather(g0 + b + NBUF, b)
            return carry

        lax.fori_loop(0, n_outer - 1, outer, 0)

        g0 = (n_outer - 1) * NBUF
        for b in range(NBUF):
            gather_wait(b)
            scatter(g0 + b, b)

    return lookup


def kernel(input_ids, embed_table):
    b, l = input_ids.shape
    n = b * l
    ids = input_ids.astype(jnp.int32).reshape(NW, n // (NW * G), G)
    out = _make_lookup(n, embed_table.shape[1])(ids, embed_table)
    return out.reshape(b, l, HIDDEN)
